# baseline (speedup 1.0000x reference)
import functools
import jax
import jax.numpy as jnp
from jax import lax
from jax.experimental import pallas as pl
from jax.experimental.pallas import tpu as pltpu
from jax.experimental.pallas import tpu_sc as plsc

_CHUNK_ROWS = 32


def kernel(x, W):
    seq_len = x.shape[1]
    n_model = W.shape[1]
    info = plsc.get_sparse_core_info()
    nw = info.num_cores * info.num_subcores
    rows_per_w = seq_len // nw
    nch = rows_per_w // _CHUNK_ROWS
    mesh = plsc.VectorSubcoreMesh(core_axis_name="c", subcore_axis_name="s")

    @functools.partial(
        pl.kernel,
        mesh=mesh,
        out_type=jax.ShapeDtypeStruct((seq_len, n_model), W.dtype),
        scratch_types=[
            pltpu.VMEM((_CHUNK_ROWS, n_model), W.dtype),
            pltpu.VMEM((_CHUNK_ROWS, n_model), W.dtype),
            pltpu.SemaphoreType.DMA,
            pltpu.SemaphoreType.DMA,
            pltpu.SemaphoreType.DMA,
            pltpu.SemaphoreType.DMA,
        ],
    )
    def sc_copy(w_hbm, out_hbm, buf0, buf1, isem0, isem1, osem0, osem1):
        wid = lax.axis_index("s") * info.num_cores + lax.axis_index("c")
        base = wid * rows_per_w
        bufs = (buf0, buf1)
        isems = (isem0, isem1)
        osems = (osem0, osem1)
        in_cps = []
        out_cps = []
        for j in range(nch):
            b = j % 2
            src = w_hbm.at[pl.ds(base + j * _CHUNK_ROWS, _CHUNK_ROWS), :]
            dst = out_hbm.at[pl.ds(base + j * _CHUNK_ROWS, _CHUNK_ROWS), :]
            in_cps.append(pltpu.make_async_copy(src, bufs[b], isems[b]))
            out_cps.append(pltpu.make_async_copy(bufs[b], dst, osems[b]))
        for j in range(nch):
            b = j % 2
            if j >= 2:
                out_cps[j - 2].wait()
            in_cps[j].start()
            in_cps[j].wait()
            out_cps[j].start()
        for j in range(max(0, nch - 2), nch):
            out_cps[j].wait()

    return sc_copy(W)
